# baseline (speedup 1.0000x reference)
import functools

import jax
import jax.numpy as jnp
from jax import lax
from jax.experimental import pallas as pl
from jax.experimental.pallas import tpu as pltpu
from jax.experimental.pallas import tpu_sc as plsc

NU = 50000
EMB = 64
H = 32
NNZ = 800000
B = 1024
TEMP = 0.2
REG_L = 1e-4
SSL_L = 0.1

NTILES = 16
K = 80
NCHUNK = 626
EPT = NCHUNK * K
NNZ_PAD = NTILES * EPT
NBUF = 3
RPT = 3128
NUP = NTILES * RPT
ZR = 184


_DNUM = lax.GatherDimensionNumbers(
    offset_dims=(), collapsed_slice_dims=(0,), start_index_map=(0,))


def _spmm4(ue0bf, ie0bf, edges_u, edges_i):
  mesh = plsc.VectorSubcoreMesh(core_axis_name="c", subcore_axis_name="s")

  @functools.partial(
      pl.kernel,
      out_type=[jax.ShapeDtypeStruct((2 * NU, H), jnp.float32)] * 4,
      mesh=mesh,
      compiler_params=pltpu.CompilerParams(use_tc_tiling_on_sc=False,
                                           needs_layout_passes=False),
      scratch_types=[
          [pltpu.HBM((2 * NU, H), jnp.bfloat16)] * 2,
          pltpu.VMEM_SHARED((NUP, H), jnp.float32),
          pltpu.VMEM((ZR, H), jnp.float32),
          pltpu.VMEM((ZR, H), jnp.float32),
          pltpu.VMEM((ZR, H), jnp.bfloat16),
          [pltpu.VMEM((1, 3 * K), jnp.int32)] * NBUF,
          [pltpu.VMEM((K,), jnp.int32)] * NBUF,
          [pltpu.VMEM((K,), jnp.int32)] * NBUF,
          [pltpu.VMEM((K, H), jnp.bfloat16)] * NBUF,
          [pltpu.VMEM((K, H), jnp.float32)] * NBUF,
          [pltpu.SemaphoreType.DMA] * NBUF,
          [pltpu.SemaphoreType.DMA] * NBUF,
          [pltpu.SemaphoreType.DMA] * NBUF,
      ],
  )
  def k(ue0_h, ie0_h, eu_h, ei_h, zu1_h, zi1_h, zu2_h, zi2_h,
        zbf, acc, zbuf, fbuf, bfbuf, edge, gidx, sidx, embg, emb,
        esem, gsem, ssem):
    zu1bf_h, zi1bf_h = zbf
    c = lax.axis_index("c")
    sid = lax.axis_index("s")
    crow0 = sid * NCHUNK
    off = c * NU

    def zb(i, carry):
      zbuf[i, pl.ds(0, 16)] = jnp.zeros((16,), jnp.float32)
      zbuf[i, pl.ds(16, 16)] = jnp.zeros((16,), jnp.float32)
      return carry
    lax.fori_loop(0, ZR, zb, 0)

    def phase(table_h, edges_h, out_h, out_bf_h):
      def zc(t, carry):
        pltpu.sync_copy(zbuf, acc.at[pl.ds(sid * RPT + t * ZR, ZR)])
        return carry
      lax.fori_loop(0, RPT // ZR, zc, 0)
      plsc.subcore_barrier()

      def edge_start(i, b):
        pltpu.async_copy(edges_h.at[pl.ds(crow0 + i, 1)], edge[b], esem[b])

      def edge_wait(b):
        pltpu.make_async_copy(edges_h.at[pl.ds(0, 1)], edge[b],
                              esem[b]).wait()

      def transform(b):
        for j in range(K // 16):
          sl = pl.ds(j * 16, 16)
          sidx[b][sl] = edge[b][0, pl.ds(j * 16, 16)]
          gidx[b][sl] = edge[b][0, pl.ds(K + j * 16, 16)] + off

      def gather_start(b):
        pltpu.async_copy(table_h.at[gidx[b]], embg[b], gsem[b])

      def gather_wait(b):
        pltpu.make_async_copy(table_h.at[gidx[b]], embg[b], gsem[b]).wait()

      def scale(b):
        for g in range(K // 16):
          v16 = lax.bitcast_convert_type(
              edge[b][0, pl.ds(2 * K + g * 16, 16)], jnp.float32)
          for l in range(16):
            e = g * 16 + l
            ve = lax.gather(v16, jnp.full((16, 1), l, jnp.int32), _DNUM,
                            (1,),
                            mode=lax.GatherScatterMode.PROMISE_IN_BOUNDS)
            lo, hi = plsc.unpack(embg[b][e, :],
                                 format=plsc.PackFormat.INTERLEAVED)
            emb[b][e, pl.ds(0, 16)] = lo * ve
            emb[b][e, pl.ds(16, 16)] = hi * ve

      def scatter_start(b):
        pltpu.async_copy(emb[b], acc.at[sidx[b]], ssem[b], add=True)

      def scatter_wait(b):
        pltpu.make_async_copy(emb[b], acc.at[sidx[b]], ssem[b]).wait()

      for b in range(NBUF):
        edge_start(b, b)
      for b in range(2):
        edge_wait(b)
        transform(b)
        gather_start(b)

      def outer(g, carry):
        for u in range(NBUF):
          i = NBUF * g + u
          b = u
          p = (u + 2) % NBUF
          edge_wait(p)
          if u < 2:
            @pl.when(g > 0)
            def _():
              scatter_wait(p)
          else:
            scatter_wait(p)
          transform(p)
          gather_start(p)
          gather_wait(b)
          scale(b)
          scatter_start(b)
          @pl.when(i + NBUF < NCHUNK)
          def _():
            edge_start(i + NBUF, b)
        return carry
      lax.fori_loop(0, (NCHUNK - 2) // NBUF, outer, 0)

      for b in range(2):
        gather_wait(b)
        scale(b)
        scatter_start(b)
      for b in range(2, NBUF):
        scatter_wait(b)
      for b in range(2):
        scatter_wait(b)

      plsc.subcore_barrier()
      LASTR = NU - (NTILES - 1) * RPT

      @pl.when(sid < NTILES - 1)
      def _():
        pltpu.sync_copy(acc.at[pl.ds(sid * RPT, RPT)],
                        out_h.at[pl.ds(off + sid * RPT, RPT)])

      @pl.when(sid == NTILES - 1)
      def _():
        pltpu.sync_copy(acc.at[pl.ds((NTILES - 1) * RPT, LASTR)],
                        out_h.at[pl.ds(off + (NTILES - 1) * RPT, LASTR)])

      if out_bf_h is not None:
        def bf_chunk(base, nr):
          pltpu.sync_copy(acc.at[pl.ds(base, nr)], fbuf.at[pl.ds(0, nr)])

          def rowloop(r, carry):
            bfbuf[r, :] = plsc.pack(fbuf[r, pl.ds(0, 16)],
                                    fbuf[r, pl.ds(16, 16)],
                                    format=plsc.PackFormat.INTERLEAVED)
            return carry
          lax.fori_loop(0, nr, rowloop, 0)
          pltpu.sync_copy(bfbuf.at[pl.ds(0, nr)],
                          out_bf_h.at[pl.ds(off + base, nr)])

        @pl.when(sid < NTILES - 1)
        def _():
          def step(t, carry):
            bf_chunk(sid * RPT + t * ZR, ZR)
            return carry
          lax.fori_loop(0, RPT // ZR, step, 0)

        @pl.when(sid == NTILES - 1)
        def _():
          def step(t, carry):
            bf_chunk((NTILES - 1) * RPT + t * ZR, ZR)
            return carry
          lax.fori_loop(0, LASTR // ZR, step, 0)
          bf_chunk((NTILES - 1) * RPT + (LASTR // ZR) * ZR,
                   LASTR - (LASTR // ZR) * ZR)
      plsc.subcore_barrier()

    phase(ie0_h, eu_h, zu1_h, zu1bf_h)
    phase(ue0_h, ei_h, zi1_h, zi1bf_h)
    phase(zi1bf_h, eu_h, zu2_h, None)
    phase(zu1bf_h, ei_h, zi2_h, None)

  return k(ue0bf, ie0bf, edges_u, edges_i)


def _sc_gather(tabs, cat_idx):
  mesh = plsc.VectorSubcoreMesh(core_axis_name="c", subcore_axis_name="s")
  JOB_TAB = [0, 1, 2, 3, 4, 5, 3, 5]

  @functools.partial(
      pl.kernel,
      out_type=jax.ShapeDtypeStruct((16 * B, H), jnp.float32),
      mesh=mesh,
      compiler_params=pltpu.CompilerParams(use_tc_tiling_on_sc=False),
      scratch_types=[
          pltpu.VMEM((128,), jnp.int32),
          pltpu.VMEM((128, H), jnp.float32),
          pltpu.SemaphoreType.DMA,
      ],
  )
  def k(t0, t1, t2, t3, t4, t5, idx_h, out_h, idx_v, emb_v, sem):
    c = lax.axis_index("c")
    sid = lax.axis_index("s")
    w = sid * 2 + c
    j = w // 4
    h = (w // 2) % 2
    r0 = (w % 2) * 512
    trefs = [t0, t1, t2, t3, t4, t5]

    for jj in range(8):
      @pl.when(j == jj)
      def _(jj=jj):
        tab = trefs[JOB_TAB[jj]]
        for kk in range(4):
          pltpu.sync_copy(idx_h.at[pl.ds(jj * B + r0 + kk * 128, 128)],
                          idx_v)
          for t in range(8):
            sl = pl.ds(t * 16, 16)
            idx_v[sl] = idx_v[sl] + h * NU
          pltpu.async_copy(tab.at[idx_v], emb_v, sem).wait()
          pltpu.sync_copy(
              emb_v,
              out_h.at[pl.ds((jj * 2 + h) * B + r0 + kk * 128, 128)])

  return k(*tabs, cat_idx)


BLK = 2000
NB = NU // BLK


def _tc_small(svd_u, svd_v, ue3, ie3):
  def body(su_ref, sv_ref, ue_ref, ie_ref, utu_ref, vti_ref):
    j = pl.program_id(0)

    @pl.when(j == 0)
    def _():
      utu_ref[...] = jnp.zeros_like(utu_ref)
      vti_ref[...] = jnp.zeros_like(vti_ref)

    su = su_ref[...]
    sv = sv_ref[...]
    dn = (((0,), (0,)), ((), ()))
    for c in range(2):
      utu_ref[c] += lax.dot_general(su, ue_ref[c], dn,
                                    preferred_element_type=jnp.float32)
      vti_ref[c] += lax.dot_general(sv, ie_ref[c], dn,
                                    preferred_element_type=jnp.float32)

  return pl.pallas_call(
      body,
      grid=(NB,),
      in_specs=[
          pl.BlockSpec((BLK, EMB), lambda j: (j, 0)),
          pl.BlockSpec((BLK, EMB), lambda j: (j, 0)),
          pl.BlockSpec((2, BLK, H), lambda j: (0, j, 0)),
          pl.BlockSpec((2, BLK, H), lambda j: (0, j, 0)),
      ],
      out_specs=[
          pl.BlockSpec((2, EMB, H), lambda j: (0, 0, 0)),
          pl.BlockSpec((2, EMB, H), lambda j: (0, 0, 0)),
      ],
      out_shape=[jax.ShapeDtypeStruct((2, EMB, H), jnp.float32)] * 2,
  )(svd_u, svd_v, ue3, ie3)


def _tc_big(svd_u, svd_v, s2, utu, vti, gsu3, gsi3, smu3, smi3, zu3, zi3):
  def body(su_ref, sv_ref, s_ref, utu_ref, vti_ref, gsu_ref, gsi_ref,
           smu_ref, smi_ref, zu_ref, zi_ref, gsu_o, gsi_o, smu_o, smi_o):
    s = s_ref[...]
    um = su_ref[...] * s
    vm = sv_ref[...] * s
    for c in range(2):
      g_u = jnp.dot(um, vti_ref[c], preferred_element_type=jnp.float32)
      g_i = jnp.dot(vm, utu_ref[c], preferred_element_type=jnp.float32)
      gsu_o[c] = gsu_ref[c] + g_u
      gsi_o[c] = gsi_ref[c] + g_i
      smu_o[c] = smu_ref[c] + zu_ref[c]
      smi_o[c] = smi_ref[c] + zi_ref[c]

  blk3 = pl.BlockSpec((2, BLK, H), lambda j: (0, j, 0))
  return pl.pallas_call(
      body,
      grid=(NB,),
      in_specs=[
          pl.BlockSpec((BLK, EMB), lambda j: (j, 0)),
          pl.BlockSpec((BLK, EMB), lambda j: (j, 0)),
          pl.BlockSpec((1, EMB), lambda j: (0, 0)),
          pl.BlockSpec((2, EMB, H), lambda j: (0, 0, 0)),
          pl.BlockSpec((2, EMB, H), lambda j: (0, 0, 0)),
          blk3, blk3, blk3, blk3, blk3, blk3,
      ],
      out_specs=[blk3, blk3, blk3, blk3],
      out_shape=[jax.ShapeDtypeStruct((2, NU, H), jnp.float32)] * 4,
  )(svd_u, svd_v, s2, utu, vti, gsu3, gsi3, smu3, smi3, zu3, zi3)


def _tc_final(G, au3, ai3):
  def body(g_ref, au_ref, ai_ref, bpr_ref, reg_ref, ssl_ref,
           su_acc, si_acc):
    j = pl.program_id(0)

    @pl.when(j == 0)
    def _():
      su_acc[...] = jnp.zeros_like(su_acc)
      si_acc[...] = jnp.zeros_like(si_acc)

    gu_sel = g_ref[1]
    gi_sel = g_ref[4]
    dn = (((1,), (1,)), ((), ()))
    lu = lax.dot_general(gu_sel[:, 0:H], au_ref[0], dn,
                         preferred_element_type=jnp.float32)
    lu += lax.dot_general(gu_sel[:, H:EMB], au_ref[1], dn,
                          preferred_element_type=jnp.float32)
    li = lax.dot_general(gi_sel[:, 0:H], ai_ref[0], dn,
                         preferred_element_type=jnp.float32)
    li += lax.dot_general(gi_sel[:, H:EMB], ai_ref[1], dn,
                          preferred_element_type=jnp.float32)
    su_acc[...] += jnp.sum(jnp.exp(lu / TEMP), axis=1, keepdims=True)
    si_acc[...] += jnp.sum(jnp.exp(li / TEMP), axis=1, keepdims=True)

    @pl.when(j == NB - 1)
    def _():
      u_e = g_ref[0]
      ego_u = g_ref[2]
      p_e = g_ref[3]
      ego_p = g_ref[5]
      n_e = g_ref[6]
      ego_n = g_ref[7]
      pos_scores = jnp.sum(u_e * p_e, axis=1)
      neg_scores = jnp.sum(u_e * n_e, axis=1)
      x = pos_scores - neg_scores
      sp = jnp.maximum(-x, 0.0) + jnp.log(1.0 + jnp.exp(-jnp.abs(x)))
      bpr_ref[0, 0] = jnp.mean(sp)
      reg_ref[0, 0] = REG_L * 0.5 * (
          jnp.sum(ego_u ** 2) + jnp.sum(ego_p ** 2) + jnp.sum(ego_n ** 2)
      ) / B
      neg_sc = (jnp.mean(jnp.log(su_acc[...] + 1e-8))
                + jnp.mean(jnp.log(si_acc[...] + 1e-8)))
      pos_sc = (jnp.mean(jnp.clip(jnp.sum(u_e * gu_sel, axis=1) / TEMP,
                                  -5.0, 5.0))
                + jnp.mean(jnp.clip(jnp.sum(p_e * gi_sel, axis=1) / TEMP,
                                    -5.0, 5.0)))
      ssl_ref[0, 0] = SSL_L * (-pos_sc + neg_sc)

  blk1 = pl.BlockSpec((1, 1), lambda j: (0, 0), memory_space=pltpu.SMEM)
  outs = pl.pallas_call(
      body,
      grid=(NB,),
      in_specs=[
          pl.BlockSpec((8, B, EMB), lambda j: (0, 0, 0)),
          pl.BlockSpec((2, BLK, H), lambda j: (0, j, 0)),
          pl.BlockSpec((2, BLK, H), lambda j: (0, j, 0)),
      ],
      out_specs=[blk1, blk1, blk1],
      out_shape=[jax.ShapeDtypeStruct((1, 1), jnp.float32)] * 3,
      scratch_shapes=[
          pltpu.VMEM((B, 1), jnp.float32),
          pltpu.VMEM((B, 1), jnp.float32),
      ],
  )(G, au3, ai3)
  return outs


def kernel(user_emb_w, item_emb_w, graph_vals, svd_u, s, svd_v, user,
           positive, negative, graph_rows, graph_cols):
  ue0f = user_emb_w.reshape(NU, 2, H).transpose(1, 0, 2).reshape(2 * NU, H)
  ie0f = item_emb_w.reshape(NU, 2, H).transpose(1, 0, 2).reshape(2 * NU, H)
  pad = NNZ_PAD - NNZ
  zpad = jnp.zeros((pad,), jnp.int32)
  rows = jnp.concatenate([graph_rows.astype(jnp.int32), zpad]).reshape(-1, K)
  cols = jnp.concatenate([graph_cols.astype(jnp.int32), zpad]).reshape(-1, K)
  vbits = jnp.concatenate([
      lax.bitcast_convert_type(graph_vals.astype(jnp.float32), jnp.int32),
      zpad]).reshape(-1, K)
  edges_u = jnp.concatenate([rows, cols, vbits], axis=1)
  edges_i = jnp.concatenate([cols, rows, vbits], axis=1)
  s2 = s.reshape(1, EMB)

  uef, ief = ue0f, ie0f
  gsu3 = ue0f.reshape(2, NU, H)
  gsi3 = ie0f.reshape(2, NU, H)
  smu3 = gsu3
  smi3 = gsi3

  perm = jnp.stack([jnp.arange(16), jnp.arange(16) + 16], axis=1).reshape(-1)
  ue0bf = jnp.take(ue0f, perm, axis=1).astype(jnp.bfloat16)
  ie0bf = jnp.take(ie0f, perm, axis=1).astype(jnp.bfloat16)
  zs = _spmm4(ue0bf, ie0bf, edges_u, edges_i)
  for l in range(2):
    zuf, zif = zs[2 * l], zs[2 * l + 1]
    utu, vti = _tc_small(svd_u, svd_v, uef.reshape(2, NU, H),
                         ief.reshape(2, NU, H))
    gsu3, gsi3, smu3, smi3 = _tc_big(
        svd_u, svd_v, s2, utu, vti, gsu3, gsi3, smu3, smi3,
        zuf.reshape(2, NU, H), zif.reshape(2, NU, H))
    uef, ief = zuf, zif

  cat_idx = jnp.concatenate([
      user, user, user, positive, positive, positive, negative, negative,
  ]).astype(jnp.int32)
  tabs = [smu3.reshape(2 * NU, H), gsu3.reshape(2 * NU, H), ue0f,
          smi3.reshape(2 * NU, H), gsi3.reshape(2 * NU, H), ie0f]
  gflat = _sc_gather(tabs, cat_idx)
  G = gflat.reshape(8, 2, B, H).transpose(0, 2, 1, 3).reshape(8, B, EMB)

  bpr, reg, ssl = _tc_final(G, smu3, smi3)
  return (bpr[0, 0], reg[0, 0], ssl[0, 0])


# speedup vs baseline: 1.4244x; 1.4244x over previous
import functools

import jax
import jax.numpy as jnp
from jax import lax
from jax.experimental import pallas as pl
from jax.experimental.pallas import tpu as pltpu
from jax.experimental.pallas import tpu_sc as plsc

NU = 50000
EMB = 64
H = 32
NNZ = 800000
B = 1024
TEMP = 0.2
REG_L = 1e-4
SSL_L = 0.1

NTILES = 16
K = 80
NCHUNK = 626
EPT = NCHUNK * K
NNZ_PAD = NTILES * EPT
NBUF = 4
RPT = 3128
NUP = NTILES * RPT
ZR = 184


_DNUM = lax.GatherDimensionNumbers(
    offset_dims=(), collapsed_slice_dims=(0,), start_index_map=(0,))


def _spmm4(ue0f, ie0f, edges_u, edges_i):
  mesh = plsc.VectorSubcoreMesh(core_axis_name="c", subcore_axis_name="s")

  @functools.partial(
      pl.kernel,
      out_type=[jax.ShapeDtypeStruct((2 * NU, H), jnp.float32)] * 4,
      mesh=mesh,
      compiler_params=pltpu.CompilerParams(use_tc_tiling_on_sc=False),
      scratch_types=[
          pltpu.VMEM_SHARED((NUP, H), jnp.float32),
          pltpu.VMEM((ZR, H), jnp.float32),
          [pltpu.VMEM((1, 3 * K), jnp.int32)] * NBUF,
          [pltpu.VMEM((K,), jnp.int32)] * NBUF,
          [pltpu.VMEM((K,), jnp.int32)] * NBUF,
          [pltpu.VMEM((K, H), jnp.float32)] * NBUF,
          [pltpu.SemaphoreType.DMA] * NBUF,
          [pltpu.SemaphoreType.DMA] * NBUF,
          [pltpu.SemaphoreType.DMA] * NBUF,
      ],
  )
  def k(ue0_h, ie0_h, eu_h, ei_h, zu1_h, zi1_h, zu2_h, zi2_h,
        acc, zbuf, edge, gidx, sidx, emb, esem, gsem, ssem):
    c = lax.axis_index("c")
    sid = lax.axis_index("s")
    crow0 = sid * NCHUNK
    off = c * NU

    def zb(i, carry):
      zbuf[i, pl.ds(0, 16)] = jnp.zeros((16,), jnp.float32)
      zbuf[i, pl.ds(16, 16)] = jnp.zeros((16,), jnp.float32)
      return carry
    lax.fori_loop(0, ZR, zb, 0)

    def phase(table_h, edges_h, out_h):
      def zc(t, carry):
        pltpu.sync_copy(zbuf, acc.at[pl.ds(sid * RPT + t * ZR, ZR)])
        return carry
      lax.fori_loop(0, RPT // ZR, zc, 0)
      plsc.subcore_barrier()

      def edge_start(i, b):
        pltpu.async_copy(edges_h.at[pl.ds(crow0 + i, 1)], edge[b], esem[b])

      def edge_wait(b):
        pltpu.make_async_copy(edges_h.at[pl.ds(0, 1)], edge[b],
                              esem[b]).wait()

      def transform(b):
        for j in range(K // 16):
          sl = pl.ds(j * 16, 16)
          sidx[b][sl] = edge[b][0, pl.ds(j * 16, 16)]
          gidx[b][sl] = edge[b][0, pl.ds(K + j * 16, 16)] + off

      def gather_start(b):
        pltpu.async_copy(table_h.at[gidx[b]], emb[b], gsem[b])

      def gather_wait(b):
        pltpu.make_async_copy(table_h.at[gidx[b]], emb[b], gsem[b]).wait()

      def scale(b):
        for g in range(K // 16):
          v16 = lax.bitcast_convert_type(
              edge[b][0, pl.ds(2 * K + g * 16, 16)], jnp.float32)
          for l in range(16):
            e = g * 16 + l
            ve = lax.gather(v16, jnp.full((16, 1), l, jnp.int32), _DNUM,
                            (1,),
                            mode=lax.GatherScatterMode.PROMISE_IN_BOUNDS)
            emb[b][e, pl.ds(0, 16)] = emb[b][e, pl.ds(0, 16)] * ve
            emb[b][e, pl.ds(16, 16)] = emb[b][e, pl.ds(16, 16)] * ve

      def scatter_start(b):
        pltpu.async_copy(emb[b], acc.at[sidx[b]], ssem[b], add=True)

      def scatter_wait(b):
        pltpu.make_async_copy(emb[b], acc.at[sidx[b]], ssem[b]).wait()

      for b in range(NBUF):
        edge_start(b, b)
      for b in range(2):
        edge_wait(b)
        transform(b)
        gather_start(b)

      def outer(g, carry):
        for u in range(NBUF):
          i = NBUF * g + u
          b = u
          p = (u + 2) % NBUF
          edge_wait(p)
          if u < 2:
            @pl.when(g > 0)
            def _():
              scatter_wait(p)
          else:
            scatter_wait(p)
          transform(p)
          gather_start(p)
          gather_wait(b)
          scale(b)
          scatter_start(b)
          @pl.when(i + NBUF < NCHUNK)
          def _():
            edge_start(i + NBUF, b)
        return carry
      lax.fori_loop(0, (NCHUNK - 2) // NBUF, outer, 0)

      for b in range(2):
        gather_wait(b)
        scale(b)
        scatter_start(b)
      for b in range(2, NBUF):
        scatter_wait(b)
      for b in range(2):
        scatter_wait(b)

      plsc.subcore_barrier()
      LASTR = NU - (NTILES - 1) * RPT

      @pl.when(sid < NTILES - 1)
      def _():
        pltpu.sync_copy(acc.at[pl.ds(sid * RPT, RPT)],
                        out_h.at[pl.ds(off + sid * RPT, RPT)])

      @pl.when(sid == NTILES - 1)
      def _():
        pltpu.sync_copy(acc.at[pl.ds((NTILES - 1) * RPT, LASTR)],
                        out_h.at[pl.ds(off + (NTILES - 1) * RPT, LASTR)])
      plsc.subcore_barrier()

    phase(ie0_h, eu_h, zu1_h)
    phase(ue0_h, ei_h, zi1_h)
    phase(zi1_h, eu_h, zu2_h)
    phase(zu1_h, ei_h, zi2_h)

  return k(ue0f, ie0f, edges_u, edges_i)


def _sc_gather(tabs, cat_idx):
  mesh = plsc.VectorSubcoreMesh(core_axis_name="c", subcore_axis_name="s")
  JOB_TAB = [0, 1, 2, 3, 4, 5, 3, 5]

  @functools.partial(
      pl.kernel,
      out_type=jax.ShapeDtypeStruct((16 * B, H), jnp.float32),
      mesh=mesh,
      compiler_params=pltpu.CompilerParams(use_tc_tiling_on_sc=False),
      scratch_types=[
          pltpu.VMEM((128,), jnp.int32),
          pltpu.VMEM((128, H), jnp.float32),
          pltpu.SemaphoreType.DMA,
      ],
  )
  def k(t0, t1, t2, t3, t4, t5, idx_h, out_h, idx_v, emb_v, sem):
    c = lax.axis_index("c")
    sid = lax.axis_index("s")
    w = sid * 2 + c
    j = w // 4
    h = (w // 2) % 2
    r0 = (w % 2) * 512
    trefs = [t0, t1, t2, t3, t4, t5]

    for jj in range(8):
      @pl.when(j == jj)
      def _(jj=jj):
        tab = trefs[JOB_TAB[jj]]
        for kk in range(4):
          pltpu.sync_copy(idx_h.at[pl.ds(jj * B + r0 + kk * 128, 128)],
                          idx_v)
          for t in range(8):
            sl = pl.ds(t * 16, 16)
            idx_v[sl] = idx_v[sl] + h * NU
          pltpu.async_copy(tab.at[idx_v], emb_v, sem).wait()
          pltpu.sync_copy(
              emb_v,
              out_h.at[pl.ds((jj * 2 + h) * B + r0 + kk * 128, 128)])

  return k(*tabs, cat_idx)


BLK = 2000
NB = NU // BLK


def _tc_small(svd_u, svd_v, ue3, ie3):
  def body(su_ref, sv_ref, ue_ref, ie_ref, utu_ref, vti_ref):
    j = pl.program_id(0)

    @pl.when(j == 0)
    def _():
      utu_ref[...] = jnp.zeros_like(utu_ref)
      vti_ref[...] = jnp.zeros_like(vti_ref)

    su = su_ref[...]
    sv = sv_ref[...]
    dn = (((0,), (0,)), ((), ()))
    for c in range(2):
      utu_ref[c] += lax.dot_general(su, ue_ref[c], dn,
                                    preferred_element_type=jnp.float32)
      vti_ref[c] += lax.dot_general(sv, ie_ref[c], dn,
                                    preferred_element_type=jnp.float32)

  return pl.pallas_call(
      body,
      grid=(NB,),
      in_specs=[
          pl.BlockSpec((BLK, EMB), lambda j: (j, 0)),
          pl.BlockSpec((BLK, EMB), lambda j: (j, 0)),
          pl.BlockSpec((2, BLK, H), lambda j: (0, j, 0)),
          pl.BlockSpec((2, BLK, H), lambda j: (0, j, 0)),
      ],
      out_specs=[
          pl.BlockSpec((2, EMB, H), lambda j: (0, 0, 0)),
          pl.BlockSpec((2, EMB, H), lambda j: (0, 0, 0)),
      ],
      out_shape=[jax.ShapeDtypeStruct((2, EMB, H), jnp.float32)] * 2,
  )(svd_u, svd_v, ue3, ie3)


def _tc_big(svd_u, svd_v, s2, utu, vti, gsu3, gsi3, smu3, smi3, zu3, zi3):
  def body(su_ref, sv_ref, s_ref, utu_ref, vti_ref, gsu_ref, gsi_ref,
           smu_ref, smi_ref, zu_ref, zi_ref, gsu_o, gsi_o, smu_o, smi_o):
    s = s_ref[...]
    um = su_ref[...] * s
    vm = sv_ref[...] * s
    for c in range(2):
      g_u = jnp.dot(um, vti_ref[c], preferred_element_type=jnp.float32)
      g_i = jnp.dot(vm, utu_ref[c], preferred_element_type=jnp.float32)
      gsu_o[c] = gsu_ref[c] + g_u
      gsi_o[c] = gsi_ref[c] + g_i
      smu_o[c] = smu_ref[c] + zu_ref[c]
      smi_o[c] = smi_ref[c] + zi_ref[c]

  blk3 = pl.BlockSpec((2, BLK, H), lambda j: (0, j, 0))
  return pl.pallas_call(
      body,
      grid=(NB,),
      in_specs=[
          pl.BlockSpec((BLK, EMB), lambda j: (j, 0)),
          pl.BlockSpec((BLK, EMB), lambda j: (j, 0)),
          pl.BlockSpec((1, EMB), lambda j: (0, 0)),
          pl.BlockSpec((2, EMB, H), lambda j: (0, 0, 0)),
          pl.BlockSpec((2, EMB, H), lambda j: (0, 0, 0)),
          blk3, blk3, blk3, blk3, blk3, blk3,
      ],
      out_specs=[blk3, blk3, blk3, blk3],
      out_shape=[jax.ShapeDtypeStruct((2, NU, H), jnp.float32)] * 4,
  )(svd_u, svd_v, s2, utu, vti, gsu3, gsi3, smu3, smi3, zu3, zi3)


def _tc_final(G, au3, ai3):
  def body(g_ref, au_ref, ai_ref, bpr_ref, reg_ref, ssl_ref,
           su_acc, si_acc):
    j = pl.program_id(0)

    @pl.when(j == 0)
    def _():
      su_acc[...] = jnp.zeros_like(su_acc)
      si_acc[...] = jnp.zeros_like(si_acc)

    gu_sel = g_ref[1]
    gi_sel = g_ref[4]
    dn = (((1,), (1,)), ((), ()))
    lu = lax.dot_general(gu_sel[:, 0:H], au_ref[0], dn,
                         preferred_element_type=jnp.float32)
    lu += lax.dot_general(gu_sel[:, H:EMB], au_ref[1], dn,
                          preferred_element_type=jnp.float32)
    li = lax.dot_general(gi_sel[:, 0:H], ai_ref[0], dn,
                         preferred_element_type=jnp.float32)
    li += lax.dot_general(gi_sel[:, H:EMB], ai_ref[1], dn,
                          preferred_element_type=jnp.float32)
    su_acc[...] += jnp.sum(jnp.exp(lu / TEMP), axis=1, keepdims=True)
    si_acc[...] += jnp.sum(jnp.exp(li / TEMP), axis=1, keepdims=True)

    @pl.when(j == NB - 1)
    def _():
      u_e = g_ref[0]
      ego_u = g_ref[2]
      p_e = g_ref[3]
      ego_p = g_ref[5]
      n_e = g_ref[6]
      ego_n = g_ref[7]
      pos_scores = jnp.sum(u_e * p_e, axis=1)
      neg_scores = jnp.sum(u_e * n_e, axis=1)
      x = pos_scores - neg_scores
      sp = jnp.maximum(-x, 0.0) + jnp.log(1.0 + jnp.exp(-jnp.abs(x)))
      bpr_ref[0, 0] = jnp.mean(sp)
      reg_ref[0, 0] = REG_L * 0.5 * (
          jnp.sum(ego_u ** 2) + jnp.sum(ego_p ** 2) + jnp.sum(ego_n ** 2)
      ) / B
      neg_sc = (jnp.mean(jnp.log(su_acc[...] + 1e-8))
                + jnp.mean(jnp.log(si_acc[...] + 1e-8)))
      pos_sc = (jnp.mean(jnp.clip(jnp.sum(u_e * gu_sel, axis=1) / TEMP,
                                  -5.0, 5.0))
                + jnp.mean(jnp.clip(jnp.sum(p_e * gi_sel, axis=1) / TEMP,
                                    -5.0, 5.0)))
      ssl_ref[0, 0] = SSL_L * (-pos_sc + neg_sc)

  blk1 = pl.BlockSpec((1, 1), lambda j: (0, 0), memory_space=pltpu.SMEM)
  outs = pl.pallas_call(
      body,
      grid=(NB,),
      in_specs=[
          pl.BlockSpec((8, B, EMB), lambda j: (0, 0, 0)),
          pl.BlockSpec((2, BLK, H), lambda j: (0, j, 0)),
          pl.BlockSpec((2, BLK, H), lambda j: (0, j, 0)),
      ],
      out_specs=[blk1, blk1, blk1],
      out_shape=[jax.ShapeDtypeStruct((1, 1), jnp.float32)] * 3,
      scratch_shapes=[
          pltpu.VMEM((B, 1), jnp.float32),
          pltpu.VMEM((B, 1), jnp.float32),
      ],
  )(G, au3, ai3)
  return outs


def kernel(user_emb_w, item_emb_w, graph_vals, svd_u, s, svd_v, user,
           positive, negative, graph_rows, graph_cols):
  ue0f = user_emb_w.reshape(NU, 2, H).transpose(1, 0, 2).reshape(2 * NU, H)
  ie0f = item_emb_w.reshape(NU, 2, H).transpose(1, 0, 2).reshape(2 * NU, H)
  pad = NNZ_PAD - NNZ
  zpad = jnp.zeros((pad,), jnp.int32)
  rows = jnp.concatenate([graph_rows.astype(jnp.int32), zpad]).reshape(-1, K)
  cols = jnp.concatenate([graph_cols.astype(jnp.int32), zpad]).reshape(-1, K)
  vbits = jnp.concatenate([
      lax.bitcast_convert_type(graph_vals.astype(jnp.float32), jnp.int32),
      zpad]).reshape(-1, K)
  edges_u = jnp.concatenate([rows, cols, vbits], axis=1)
  edges_i = jnp.concatenate([cols, rows, vbits], axis=1)
  s2 = s.reshape(1, EMB)

  uef, ief = ue0f, ie0f
  gsu3 = ue0f.reshape(2, NU, H)
  gsi3 = ie0f.reshape(2, NU, H)
  smu3 = gsu3
  smi3 = gsi3

  zs = _spmm4(ue0f, ie0f, edges_u, edges_i)
  for l in range(2):
    zuf, zif = zs[2 * l], zs[2 * l + 1]
    utu, vti = _tc_small(svd_u, svd_v, uef.reshape(2, NU, H),
                         ief.reshape(2, NU, H))
    gsu3, gsi3, smu3, smi3 = _tc_big(
        svd_u, svd_v, s2, utu, vti, gsu3, gsi3, smu3, smi3,
        zuf.reshape(2, NU, H), zif.reshape(2, NU, H))
    uef, ief = zuf, zif

  cat_idx = jnp.concatenate([
      user, user, user, positive, positive, positive, negative, negative,
  ]).astype(jnp.int32)
  tabs = [smu3.reshape(2 * NU, H), gsu3.reshape(2 * NU, H), ue0f,
          smi3.reshape(2 * NU, H), gsi3.reshape(2 * NU, H), ie0f]
  gflat = _sc_gather(tabs, cat_idx)
  G = gflat.reshape(8, 2, B, H).transpose(0, 2, 1, 3).reshape(8, B, EMB)

  bpr, reg, ssl = _tc_final(G, smu3, smi3)
  return (bpr[0, 0], reg[0, 0], ssl[0, 0])
